# baseline (speedup 1.0000x reference)
import functools

import jax
import jax.numpy as jnp
from jax import lax
from jax.experimental import pallas as pl
from jax.experimental.pallas import tpu as pltpu
from jax.experimental.pallas import tpu_sc as plsc

_D = 1024
_SCALE = float(_D ** 0.5)
_LANES = 16

_NC = 2
_NS = 16
_NW = _NC * _NS
_NR = 4
_NSRC = 3
_PREF = 3

_N_SC = 4096


def _sc_body(tok_per_w, chunk, idx_hbm, src_hbm, table_hbm, out_hbm,
             idx_v, *bufs):
    rows = bufs[0:_NR]
    src = bufs[_NR:_NR + _NSRC]
    o = _NR + _NSRC
    gsem = bufs[o:o + _NR]
    ssem = bufs[o + _NR:o + _NR + _NSRC]
    osem = bufs[o + _NR + _NSRC:o + 2 * _NR + _NSRC]

    wid = lax.axis_index("s") * _NC + lax.axis_index("c")
    base = wid * tok_per_w
    n_chunks = tok_per_w // chunk

    pltpu.sync_copy(idx_hbm.at[pl.ds(base, tok_per_w)], idx_v)

    def issue_in(c):
        rb, sb = c % _NR, c % _NSRC
        g = pltpu.async_copy(table_hbm.at[idx_v.at[pl.ds(c * chunk, chunk)]],
                             rows[rb], gsem[rb])
        s = pltpu.async_copy(src_hbm.at[pl.ds(base + c * chunk, chunk)],
                             src[sb], ssem[sb])
        return g, s

    in_flight = {}
    out_flight = {}
    for c in range(min(_PREF, n_chunks)):
        in_flight[c] = issue_in(c)

    for c in range(n_chunks):
        rb, sb = c % _NR, c % _NSRC
        g, s = in_flight.pop(c)
        g.wait()
        s.wait()

        def row_body(r, rcarry):
            for j in range(_D // _LANES):
                sl = pl.ds(j * _LANES, _LANES)
                plsc.addupdate(rows[rb].at[r, sl], src[sb][r, sl] * _SCALE)
            return rcarry

        lax.fori_loop(0, chunk, row_body, 0)

        out_flight[c] = pltpu.async_copy(
            rows[rb], out_hbm.at[pl.ds(base + c * chunk, chunk)], osem[rb])

        nxt = c + _PREF
        if nxt < n_chunks:
            old = nxt - _NR
            if old >= 0:
                out_flight.pop(old).wait()
            in_flight[nxt] = issue_in(nxt)

    for c in sorted(out_flight):
        out_flight.pop(c).wait()


@functools.partial(jax.jit, static_argnames=("n_tok", "chunk"))
def _sc_call(idx, src, table, n_tok, chunk):
    tok_per_w = n_tok // _NW
    mesh = plsc.VectorSubcoreMesh(core_axis_name="c", subcore_axis_name="s")
    body = functools.partial(_sc_body, tok_per_w, chunk)
    return pl.kernel(
        body,
        out_type=jax.ShapeDtypeStruct((n_tok, _D), jnp.float32),
        mesh=mesh,
        scratch_types=(
            [pltpu.VMEM((tok_per_w,), jnp.int32)]
            + [pltpu.VMEM((chunk, _D), jnp.float32)
               for _ in range(_NR + _NSRC)]
            + [pltpu.SemaphoreType.DMA for _ in range(2 * _NR + _NSRC)]
        ),
    )(idx, src, table)


def _tc_body(tb, base_blocks, idx_smem, src_ref, table_ref, out_ref):
    gi = pl.program_id(0)
    tok0 = (base_blocks + gi) * tb

    def tok_body(t, carry):
        for u in range(4):
            tt = t * 4 + u
            r = idx_smem[tok0 + tt]
            dst = pl.ds(tt * 8, 8)
            tsl = pl.ds(pl.multiple_of(r * 8, 8), 8)
            out_ref[dst, :] = src_ref[dst, :] * _SCALE + table_ref[tsl, :]
        return carry

    lax.fori_loop(0, tb // 4, tok_body, 0)


@functools.partial(jax.jit, static_argnames=("n_sc", "n_tok", "tb"))
def _tc_call(idx, src8, table8, n_sc, n_tok, tb):
    m = n_tok - n_sc
    base_blocks = n_sc // tb
    grid_spec = pltpu.PrefetchScalarGridSpec(
        num_scalar_prefetch=1,
        grid=(m // tb,),
        in_specs=[
            pl.BlockSpec((tb * 8, 128),
                         lambda i, idx_ref: (i + base_blocks, 0)),
            pl.BlockSpec(table8.shape, lambda i, idx_ref: (0, 0)),
        ],
        out_specs=pl.BlockSpec((tb * 8, 128), lambda i, idx_ref: (i, 0)),
    )
    return pl.pallas_call(
        functools.partial(_tc_body, tb, base_blocks),
        grid_spec=grid_spec,
        out_shape=jax.ShapeDtypeStruct((m * 8, 128), jnp.float32),
    )(idx, src8, table8)


def kernel(src_word, src_pos, pos_table):
    b, s, d = src_word.shape
    n_tok = b * s
    src = src_word.reshape(n_tok, d)
    idx = src_pos.reshape(n_tok)
    out_sc = _sc_call(idx, src, pos_table, _N_SC, 16)
    src8 = src_word.reshape(n_tok * 8, 128)
    table8 = pos_table.reshape(-1, 128)
    out_tc = _tc_call(idx, src8, table8, _N_SC, n_tok, 512)
    out = jnp.concatenate([out_sc.reshape(_N_SC * 8, 128), out_tc], axis=0)
    return out.reshape(b, s, d)


# speedup vs baseline: 2.6000x; 2.6000x over previous
import functools

import jax
import jax.numpy as jnp
from jax import lax
from jax.experimental import pallas as pl
from jax.experimental.pallas import tpu as pltpu
from jax.experimental.pallas import tpu_sc as plsc

_D = 1024
_SCALE = float(_D ** 0.5)
_LANES = 16

_NC = 2
_NS = 16
_NW = _NC * _NS
_NR = 4
_NSRC = 3
_PREF = 3


def _sc_body(tok_per_w, chunk, idx_hbm, src_hbm, table_hbm, out_hbm,
             idx_v, *bufs):
    rows = bufs[0:_NR]
    src = bufs[_NR:_NR + _NSRC]
    o = _NR + _NSRC
    gsem = bufs[o:o + _NR]
    ssem = bufs[o + _NR:o + _NR + _NSRC]
    osem = bufs[o + _NR + _NSRC:o + 2 * _NR + _NSRC]

    wid = lax.axis_index("s") * _NC + lax.axis_index("c")
    base = wid * tok_per_w
    n_chunks = tok_per_w // chunk

    pltpu.sync_copy(idx_hbm.at[pl.ds(base, tok_per_w)], idx_v)

    def issue_in(c):
        rb, sb = c % _NR, c % _NSRC
        g = pltpu.async_copy(table_hbm.at[idx_v.at[pl.ds(c * chunk, chunk)]],
                             rows[rb], gsem[rb])
        s = pltpu.async_copy(src_hbm.at[pl.ds(base + c * chunk, chunk)],
                             src[sb], ssem[sb])
        return g, s

    in_flight = {}
    out_flight = {}
    for c in range(min(_PREF, n_chunks)):
        in_flight[c] = issue_in(c)

    for c in range(n_chunks):
        rb, sb = c % _NR, c % _NSRC
        g, s = in_flight.pop(c)
        g.wait()
        s.wait()

        def row_body(r, rcarry):
            for j in range(_D // _LANES):
                sl = pl.ds(j * _LANES, _LANES)
                plsc.addupdate(rows[rb].at[r, sl], src[sb][r, sl] * _SCALE)
            return rcarry

        lax.fori_loop(0, chunk, row_body, 0)

        out_flight[c] = pltpu.async_copy(
            rows[rb], out_hbm.at[pl.ds(base + c * chunk, chunk)], osem[rb])

        nxt = c + _PREF
        if nxt < n_chunks:
            old = nxt - _NR
            if old >= 0:
                out_flight.pop(old).wait()
            in_flight[nxt] = issue_in(nxt)

    for c in sorted(out_flight):
        out_flight.pop(c).wait()


@functools.partial(jax.jit, static_argnames=("n_tok", "chunk"))
def _sc_call(idx, src, table, n_tok, chunk):
    tok_per_w = n_tok // _NW
    mesh = plsc.VectorSubcoreMesh(core_axis_name="c", subcore_axis_name="s")
    body = functools.partial(_sc_body, tok_per_w, chunk)
    return pl.kernel(
        body,
        out_type=jax.ShapeDtypeStruct((n_tok, _D), jnp.float32),
        mesh=mesh,
        scratch_types=(
            [pltpu.VMEM((tok_per_w,), jnp.int32)]
            + [pltpu.VMEM((chunk, _D), jnp.float32)
               for _ in range(_NR + _NSRC)]
            + [pltpu.SemaphoreType.DMA for _ in range(2 * _NR + _NSRC)]
        ),
    )(idx, src, table)


def kernel(src_word, src_pos, pos_table):
    b, s, d = src_word.shape
    n_tok = b * s
    src = src_word.reshape(n_tok, d)
    idx = src_pos.reshape(n_tok)
    out = _sc_call(idx, src, pos_table, n_tok, 16)
    return out.reshape(b, s, d)
